# baseline (speedup 1.0000x reference)
import functools

import jax
import jax.numpy as jnp
from jax import lax
from jax.experimental import pallas as pl
from jax.experimental.pallas import tpu as pltpu
from jax.experimental.pallas import tpu_sc as plsc

EMBED = 32
LANES = 16
CHUNK = 128
PROJ_BLK = 32768


def _proj_body(w_ref, tab_ref, out_ref):
    out_ref[...] = jnp.dot(
        w_ref[...], tab_ref[...], preferred_element_type=jnp.float32)[0]


def _tc_project(w_row, tab_t):
    n = tab_t.shape[1]
    grid = pl.cdiv(n, PROJ_BLK)
    return pl.pallas_call(
        _proj_body,
        grid=(grid,),
        in_specs=[
            pl.BlockSpec((1, EMBED), lambda i: (0, 0)),
            pl.BlockSpec((EMBED, PROJ_BLK), lambda i: (0, i)),
        ],
        out_specs=pl.BlockSpec((PROJ_BLK,), lambda i: (i,)),
        out_shape=jax.ShapeDtypeStruct((n,), jnp.float32),
    )(w_row, tab_t)


def _sc_body(nc, bpw, user_h, course_h, pu_h, pc_h, bv_h, out_h,
             idx_u, idx_c, g_u, g_c, bv_v, out_v, sem):
    wid = lax.axis_index("s") * nc + lax.axis_index("c")
    base = wid * bpw
    nch = bpw // CHUNK

    pltpu.sync_copy(bv_h, bv_v)
    for j in range(nch):
        pltpu.sync_copy(user_h.at[pl.ds(base + j * CHUNK, CHUNK)], idx_u.at[j])
        pltpu.sync_copy(course_h.at[pl.ds(base + j * CHUNK, CHUNK)], idx_c.at[j])

    copies = []
    for j in range(nch):
        copies.append(pltpu.async_copy(pu_h.at[idx_u.at[j]], g_u.at[j], sem))
        copies.append(pltpu.async_copy(pc_h.at[idx_c.at[j]], g_c.at[j], sem))
    for c in copies:
        c.wait()

    b_vec = bv_v[...]
    for j in range(nch):
        for m in range(0, CHUNK, LANES):
            t = g_u[j, pl.ds(m, LANES)] + g_c[j, pl.ds(m, LANES)] + b_vec
            out_v[pl.ds(j * CHUNK + m, LANES)] = t

    pltpu.sync_copy(out_v, out_h.at[pl.ds(base, bpw)])


def _sc_gather_add(user, course, proj_u, proj_c, b_vec):
    batch = user.shape[0]
    info = plsc.get_sparse_core_info()
    nc, ns = info.num_cores, info.num_subcores
    bpw = batch // (nc * ns)

    mesh = plsc.VectorSubcoreMesh(core_axis_name="c", subcore_axis_name="s")
    fn = pl.kernel(
        functools.partial(_sc_body, nc, bpw),
        out_type=jax.ShapeDtypeStruct((batch,), jnp.float32),
        mesh=mesh,
        compiler_params=pltpu.CompilerParams(
            needs_layout_passes=False, use_tc_tiling_on_sc=False),
        scratch_types=[
            pltpu.VMEM((bpw // CHUNK, CHUNK), jnp.int32),
            pltpu.VMEM((bpw // CHUNK, CHUNK), jnp.int32),
            pltpu.VMEM((bpw // CHUNK, CHUNK), jnp.float32),
            pltpu.VMEM((bpw // CHUNK, CHUNK), jnp.float32),
            pltpu.VMEM((LANES,), jnp.float32),
            pltpu.VMEM((bpw,), jnp.float32),
            pltpu.SemaphoreType.DMA,
        ],
    )
    return fn(user, course, proj_u, proj_c, b_vec)


@jax.jit
def _run(user, course, user_table, course_table, W, b):
    w_u = W[:, :EMBED]
    w_c = W[:, EMBED:]
    proj_u = _tc_project(w_u, user_table.T)
    proj_c = _tc_project(w_c, course_table.T)
    b_vec = jnp.broadcast_to(b, (LANES,)).astype(jnp.float32)
    return _sc_gather_add(user, course, proj_u, proj_c, b_vec)


def kernel(user, course, user_table, course_table, W, b):
    out = _run(user, course, user_table, course_table, W, b)
    return out.reshape(-1, 1)


# speedup vs baseline: 1.0924x; 1.0924x over previous
import functools

import jax
import jax.numpy as jnp
from jax import lax
from jax.experimental import pallas as pl
from jax.experimental.pallas import tpu as pltpu
from jax.experimental.pallas import tpu_sc as plsc

EMBED = 32
LANES = 16
CHUNK = 128
PROJ_BLK = 65536


def _proj_body(w_ref, tab_ref, out_ref):
    out_ref[...] = jnp.dot(
        w_ref[...], tab_ref[...], preferred_element_type=jnp.float32)[0]


def _tc_project(w_row, tab_t):
    n = tab_t.shape[1]
    grid = pl.cdiv(n, PROJ_BLK)
    return pl.pallas_call(
        _proj_body,
        grid=(grid,),
        in_specs=[
            pl.BlockSpec((1, EMBED), lambda i: (0, 0)),
            pl.BlockSpec((EMBED, PROJ_BLK), lambda i: (0, i)),
        ],
        out_specs=pl.BlockSpec((PROJ_BLK,), lambda i: (i,)),
        out_shape=jax.ShapeDtypeStruct((n,), jnp.float32),
    )(w_row, tab_t)


def _sc_body(nc, bpw, idx_h, proj_h, bv_h, out_h,
             idx_v, g_v, bv_v, out_v, sem):
    wid = lax.axis_index("s") * nc + lax.axis_index("c")
    base = wid * bpw
    nch = bpw // CHUNK
    elementwise = bv_h.shape[0] != LANES

    if elementwise:
        pltpu.sync_copy(bv_h.at[pl.ds(base, bpw)], bv_v)
    else:
        pltpu.sync_copy(bv_h, bv_v)
    for j in range(nch):
        pltpu.sync_copy(idx_h.at[pl.ds(base + j * CHUNK, CHUNK)], idx_v.at[j])

    copies = []
    for j in range(nch):
        copies.append(pltpu.async_copy(proj_h.at[idx_v.at[j]], g_v.at[j], sem))
    for c in copies:
        c.wait()

    for j in range(nch):
        for m in range(0, CHUNK, LANES):
            i = j * CHUNK + m
            bval = bv_v[pl.ds(i, LANES)] if elementwise else bv_v[...]
            out_v[pl.ds(i, LANES)] = g_v[j, pl.ds(m, LANES)] + bval

    pltpu.sync_copy(out_v, out_h.at[pl.ds(base, bpw)])


def _sc_gather_add(idx, proj, base_vals):
    batch = idx.shape[0]
    info = plsc.get_sparse_core_info()
    nc, ns = info.num_cores, info.num_subcores
    bpw = batch // (nc * ns)

    bv_shape = (LANES,) if base_vals.shape[0] == LANES else (bpw,)
    mesh = plsc.VectorSubcoreMesh(core_axis_name="c", subcore_axis_name="s")
    fn = pl.kernel(
        functools.partial(_sc_body, nc, bpw),
        out_type=jax.ShapeDtypeStruct((batch,), jnp.float32),
        mesh=mesh,
        compiler_params=pltpu.CompilerParams(
            needs_layout_passes=False, use_tc_tiling_on_sc=False),
        scratch_types=[
            pltpu.VMEM((bpw // CHUNK, CHUNK), jnp.int32),
            pltpu.VMEM((bpw // CHUNK, CHUNK), jnp.float32),
            pltpu.VMEM(bv_shape, jnp.float32),
            pltpu.VMEM((bpw,), jnp.float32),
            pltpu.SemaphoreType.DMA,
        ],
    )
    return fn(idx, proj, base_vals)


@jax.jit
def _run(user, course, user_table, course_table, W, b):
    w_u = W[:, :EMBED]
    w_c = W[:, EMBED:]
    b_vec = jnp.broadcast_to(b, (LANES,)).astype(jnp.float32)
    proj_c = _tc_project(w_c, course_table.T)
    partial = _sc_gather_add(course, proj_c, b_vec)
    proj_u = _tc_project(w_u, user_table.T)
    return _sc_gather_add(user, proj_u, partial)


def kernel(user, course, user_table, course_table, W, b):
    out = _run(user, course, user_table, course_table, W, b)
    return out.reshape(-1, 1)


# speedup vs baseline: 1.1239x; 1.0289x over previous
import functools

import jax
import jax.numpy as jnp
from jax import lax
from jax.experimental import pallas as pl
from jax.experimental.pallas import tpu as pltpu
from jax.experimental.pallas import tpu_sc as plsc

EMBED = 32
LANES = 16
CHUNK = 64
PROJ_BLK = 65536


def _proj_body(w_ref, tab_ref, out_ref):
    out_ref[...] = jnp.dot(
        w_ref[...], tab_ref[...], preferred_element_type=jnp.float32)[0]


def _tc_project(w_row, tab_t):
    n = tab_t.shape[1]
    grid = pl.cdiv(n, PROJ_BLK)
    return pl.pallas_call(
        _proj_body,
        grid=(grid,),
        in_specs=[
            pl.BlockSpec((1, EMBED), lambda i: (0, 0)),
            pl.BlockSpec((EMBED, PROJ_BLK), lambda i: (0, i)),
        ],
        out_specs=pl.BlockSpec((PROJ_BLK,), lambda i: (i,)),
        out_shape=jax.ShapeDtypeStruct((n,), jnp.float32),
    )(w_row, tab_t)


def _sc_body(nc, bpw, idx_h, proj_h, bv_h, out_h,
             idx_v, g_v, bv_v, out_v, sem_i, sem_b, sem):
    wid = lax.axis_index("s") * nc + lax.axis_index("c")
    base = wid * bpw
    nch = bpw // CHUNK
    elementwise = bv_h.shape[0] != LANES

    idx_cp = pltpu.async_copy(idx_h.at[pl.ds(base, bpw)], idx_v, sem_i)
    if elementwise:
        bv_cp = pltpu.async_copy(bv_h.at[pl.ds(base, bpw)], bv_v, sem_b)
    else:
        bv_cp = pltpu.async_copy(bv_h, bv_v, sem_b)

    idx_cp.wait()
    copies = []
    for j in range(nch):
        copies.append(pltpu.async_copy(
            proj_h.at[idx_v.at[pl.ds(j * CHUNK, CHUNK)]],
            g_v.at[pl.ds(j * CHUNK, CHUNK)], sem))
    bv_cp.wait()
    for c in copies:
        c.wait()

    for i in range(0, bpw, LANES):
        bval = bv_v[pl.ds(i, LANES)] if elementwise else bv_v[...]
        out_v[pl.ds(i, LANES)] = g_v[pl.ds(i, LANES)] + bval

    pltpu.sync_copy(out_v, out_h.at[pl.ds(base, bpw)])


def _sc_gather_add(idx, proj, base_vals):
    batch = idx.shape[0]
    info = plsc.get_sparse_core_info()
    nc, ns = info.num_cores, info.num_subcores
    bpw = batch // (nc * ns)

    bv_shape = (LANES,) if base_vals.shape[0] == LANES else (bpw,)
    mesh = plsc.VectorSubcoreMesh(core_axis_name="c", subcore_axis_name="s")
    fn = pl.kernel(
        functools.partial(_sc_body, nc, bpw),
        out_type=jax.ShapeDtypeStruct((batch,), jnp.float32),
        mesh=mesh,
        compiler_params=pltpu.CompilerParams(
            needs_layout_passes=False, use_tc_tiling_on_sc=False),
        scratch_types=[
            pltpu.VMEM((bpw,), jnp.int32),
            pltpu.VMEM((bpw,), jnp.float32),
            pltpu.VMEM(bv_shape, jnp.float32),
            pltpu.VMEM((bpw,), jnp.float32),
            pltpu.SemaphoreType.DMA,
            pltpu.SemaphoreType.DMA,
            pltpu.SemaphoreType.DMA,
        ],
    )
    return fn(idx, proj, base_vals)


@jax.jit
def _run(user, course, user_table, course_table, W, b):
    w_u = W[:, :EMBED]
    w_c = W[:, EMBED:]
    b_vec = jnp.broadcast_to(b, (LANES,)).astype(jnp.float32)
    proj_c = _tc_project(w_c, course_table.T)
    partial = _sc_gather_add(course, proj_c, b_vec)
    proj_u = _tc_project(w_u, user_table.T)
    return _sc_gather_add(user, proj_u, partial)


def kernel(user, course, user_table, course_table, W, b):
    out = _run(user, course, user_table, course_table, W, b)
    return out.reshape(-1, 1)
